# Initial kernel scaffold; baseline (speedup 1.0000x reference)
import jax
import jax.numpy as jnp
from jax.experimental import pallas as pl


def kernel(x, edge_index, edge_attr, u, batch, W1a, b1a, W1b, b1b, W2a, b2a, W2b, b2b):
    raise NotImplementedError("write your pallas kernel here")


# speedup vs baseline: 2.9178x; 2.9178x over previous
import functools

import jax
import jax.numpy as jnp
from jax import lax
from jax.experimental import pallas as pl
from jax.experimental.pallas import tpu as pltpu
from jax.experimental.pallas import tpu_sc as plsc

N_NODES = 10000
N_EDGES = 320000
D_FEAT = 128
D_EDGE = 16
HID = 128
N_TGT = 128

_NC, _NS = 2, 16
_C = 80
_EPT = N_EDGES // (_NC * _NS)
_NCHUNK = _EPT // _C
_WR = 80
_NWCHUNK = N_NODES // _WR
_EB = 2000
_NB = 1000


def _xw_body(x_ref, w_ref, o_ref):
    o_ref[...] = jnp.dot(x_ref[...], w_ref[...],
                         preferred_element_type=jnp.float32)


def _xw_call(x, w1a_x):
    return pl.pallas_call(
        _xw_body,
        out_shape=jax.ShapeDtypeStruct((N_NODES, HID), jnp.float32),
    )(x, w1a_x)


def _eaw_body(ea_ref, w_ref, b_ref, o_ref):
    o_ref[...] = jnp.dot(ea_ref[...], w_ref[...],
                         preferred_element_type=jnp.float32) + b_ref[...]


def _eaw_call(ea, w1a_e, b1a):
    return pl.pallas_call(
        _eaw_body,
        grid=(N_EDGES // _EB,),
        in_specs=[
            pl.BlockSpec((_EB, D_EDGE), lambda i: (i, 0)),
            pl.BlockSpec((D_EDGE, HID), lambda i: (0, 0)),
            pl.BlockSpec((1, HID), lambda i: (0, 0)),
        ],
        out_specs=pl.BlockSpec((_EB, HID), lambda i: (i, 0)),
        out_shape=jax.ShapeDtypeStruct((N_EDGES, HID), jnp.float32),
    )(ea, w1a_e, b1a.reshape(1, HID))


def _sc_edge_body(xw_hbm, eaw_hbm, col_hbm, row_hbm, sums_hbm, cnts_hbm,
                  idx_c, idx_r, eaw_v, gath_v, cnt_v, acc_sh):
    c = lax.axis_index("c")
    s = lax.axis_index("s")
    zero = jnp.zeros((16,), jnp.float32)
    one = jnp.ones((16,), jnp.float32)

    @pl.loop(0, _WR)
    def _zrow(i):
        for k in range(HID // 16):
            gath_v[i, pl.ds(k * 16, 16)] = zero

    @pl.loop(0, N_NODES // 16)
    def _zcnt(i):
        cnt_v[pl.ds(i * 16, 16)] = zero

    @pl.loop(s, _NWCHUNK, step=_NS)
    def _zinit(t):
        pltpu.sync_copy(gath_v, acc_sh.at[pl.ds(t * _WR, _WR)])
    plsc.subcore_barrier()

    base = (c * _NS + s) * _EPT

    @pl.loop(0, _NCHUNK)
    def _chunk(g):
        off = base + g * _C
        pltpu.sync_copy(col_hbm.at[pl.ds(off, _C)], idx_c)
        pltpu.sync_copy(row_hbm.at[pl.ds(off, _C)], idx_r)
        pltpu.sync_copy(eaw_hbm.at[pl.ds(off, _C)], eaw_v)
        pltpu.sync_copy(xw_hbm.at[idx_c], gath_v)

        @pl.loop(0, _C)
        def _row(r):
            for k in range(HID // 16):
                sl = pl.ds(k * 16, 16)
                gath_v[r, sl] = jnp.maximum(gath_v[r, sl] + eaw_v[r, sl], 0.0)

        @pl.loop(0, _C // 16)
        def _hist(j):
            iv = idx_r[pl.ds(j * 16, 16)]
            plsc.addupdate_scatter(cnt_v, [iv], one)

        pltpu.sync_copy(gath_v, acc_sh.at[idx_r], add=True)

    plsc.subcore_barrier()

    @pl.loop(s, _NWCHUNK, step=_NS)
    def _wout(t):
        off = t * _WR
        pltpu.sync_copy(acc_sh.at[pl.ds(off, _WR)], gath_v)
        pltpu.sync_copy(gath_v, sums_hbm.at[c, pl.ds(off, _WR)])

    pltpu.sync_copy(cnt_v, cnts_hbm.at[c * _NS + s])


_sc_edge = functools.partial(
    pl.kernel,
    out_type=(
        jax.ShapeDtypeStruct((_NC, N_NODES, HID), jnp.float32),
        jax.ShapeDtypeStruct((_NC * _NS, N_NODES), jnp.float32),
    ),
    mesh=plsc.VectorSubcoreMesh(core_axis_name="c", subcore_axis_name="s",
                                num_cores=_NC, num_subcores=_NS),
    compiler_params=pltpu.CompilerParams(needs_layout_passes=False),
    scratch_types=[
        pltpu.VMEM((_C,), jnp.int32),
        pltpu.VMEM((_C,), jnp.int32),
        pltpu.VMEM((_C, HID), jnp.float32),
        pltpu.VMEM((_C, HID), jnp.float32),
        pltpu.VMEM((N_NODES,), jnp.float32),
        pltpu.VMEM_SHARED((N_NODES, HID), jnp.float32),
    ],
)(_sc_edge_body)


def _node_body(x_ref, s_ref, c_ref, w1b_ref, b1b_ref, w2a_ref, b2a_ref,
               w2b_ref, b2b_ref, o_ref):
    ssum = s_ref[0] + s_ref[1]
    cnt = jnp.sum(c_ref[...], axis=1, keepdims=True)
    m = ssum / jnp.maximum(cnt, 1.0)
    mean = jnp.dot(m, w1b_ref[...], preferred_element_type=jnp.float32)
    mean = jnp.where(cnt > 0.0, mean + b1b_ref[...], 0.0)
    h = jnp.dot(x_ref[...], w2a_ref[:D_FEAT, :],
                preferred_element_type=jnp.float32)
    h = h + jnp.dot(mean, w2a_ref[D_FEAT:, :],
                    preferred_element_type=jnp.float32)
    h = jnp.maximum(h + b2a_ref[...], 0.0)
    o_ref[...] = jnp.dot(h, w2b_ref[...],
                         preferred_element_type=jnp.float32) + b2b_ref[...]


def _node_call(x, sums, cnts, w1b, b1b, w2a, b2a, w2b, b2b):
    nb = _NB
    return pl.pallas_call(
        _node_body,
        grid=(N_NODES // nb,),
        in_specs=[
            pl.BlockSpec((nb, D_FEAT), lambda i: (i, 0)),
            pl.BlockSpec((_NC, nb, HID), lambda i: (0, i, 0)),
            pl.BlockSpec((nb, _NC * _NS), lambda i: (i, 0)),
            pl.BlockSpec((HID, HID), lambda i: (0, 0)),
            pl.BlockSpec((1, HID), lambda i: (0, 0)),
            pl.BlockSpec((HID + D_FEAT, HID), lambda i: (0, 0)),
            pl.BlockSpec((1, HID), lambda i: (0, 0)),
            pl.BlockSpec((HID, N_TGT), lambda i: (0, 0)),
            pl.BlockSpec((1, N_TGT), lambda i: (0, 0)),
        ],
        out_specs=pl.BlockSpec((nb, N_TGT), lambda i: (i, 0)),
        out_shape=jax.ShapeDtypeStruct((N_NODES, N_TGT), jnp.float32),
    )(x, sums, cnts, w1b, b1b.reshape(1, HID), w2a, b2a.reshape(1, HID),
      w2b, b2b.reshape(1, N_TGT))


def kernel(x, edge_index, edge_attr, u, batch,
           W1a, b1a, W1b, b1b, W2a, b2a, W2b, b2b):
    ei = edge_index.astype(jnp.int32)
    row = ei[0]
    col = ei[1]
    xw = _xw_call(x, W1a[:D_FEAT])
    eaw = _eaw_call(edge_attr, W1a[D_FEAT:], b1a)
    sums, cnts = _sc_edge(xw, eaw, col, row)
    cnts_t = jnp.transpose(cnts, (1, 0))
    return _node_call(x, sums, cnts_t, W1b, b1b, W2a, b2a, W2b, b2b)


# speedup vs baseline: 3.9604x; 1.3573x over previous
import functools

import jax
import jax.numpy as jnp
from jax import lax
from jax.experimental import pallas as pl
from jax.experimental.pallas import tpu as pltpu
from jax.experimental.pallas import tpu_sc as plsc

N_NODES = 10000
N_EDGES = 320000
D_FEAT = 128
D_EDGE = 16
HID = 128
N_TGT = 128

_NC, _NS = 2, 16
_C = 64
_EPT = N_EDGES // (_NC * _NS)
_NBODY = 78
_TAIL = _EPT - 2 * _NBODY * _C
_CC = 80
_WR = 80
_NWCHUNK = N_NODES // _WR
_EB = 2000
_NB = 1000


def _xw_body(x_ref, w_ref, o_ref):
    o_ref[...] = jnp.dot(x_ref[...], w_ref[...],
                         preferred_element_type=jnp.float32)


def _xw_call(x, w1a_x):
    return pl.pallas_call(
        _xw_body,
        out_shape=jax.ShapeDtypeStruct((N_NODES, HID), jnp.float32),
    )(x, w1a_x)


def _eaw_body(ea_ref, w_ref, b_ref, o_ref):
    o_ref[...] = jnp.dot(ea_ref[...], w_ref[...],
                         preferred_element_type=jnp.float32) + b_ref[...]


def _eaw_call(ea, w1a_e, b1a):
    return pl.pallas_call(
        _eaw_body,
        grid=(N_EDGES // _EB,),
        in_specs=[
            pl.BlockSpec((_EB, D_EDGE), lambda i: (i, 0)),
            pl.BlockSpec((D_EDGE, HID), lambda i: (0, 0)),
            pl.BlockSpec((1, HID), lambda i: (0, 0)),
        ],
        out_specs=pl.BlockSpec((_EB, HID), lambda i: (i, 0)),
        out_shape=jax.ShapeDtypeStruct((N_EDGES, HID), jnp.float32),
    )(ea, w1a_e, b1a.reshape(1, HID))


def _relu_add(gath, eaw, lo, n):
    @pl.loop(lo, lo + n)
    def _row(r):
        for k in range(HID // 16):
            sl = pl.ds(k * 16, 16)
            gath[r, sl] = jnp.maximum(gath[r, sl] + eaw[r, sl], 0.0)


def _sc_edge_body(xw_hbm, eaw_hbm, col_hbm, row_hbm, sums_hbm,
                  idx_ca, idx_ra, idx_cb, idx_rb, idx_tc, idx_tr,
                  eaw_v, gath_a, gath_b, acc_sh,
                  sem_ica, sem_ira, sem_icb, sem_irb,
                  sem_ga, sem_gb, sem_e, sem_sa, sem_sb):
    c = lax.axis_index("c")
    s = lax.axis_index("s")
    zero = jnp.zeros((16,), jnp.float32)

    @pl.loop(0, _C)
    def _zrow(i):
        for k in range(HID // 16):
            gath_a[i, pl.ds(k * 16, 16)] = zero

    @pl.loop(s, N_NODES // _C, step=_NS)
    def _zinit(t):
        pltpu.sync_copy(gath_a, acc_sh.at[pl.ds(t * _C, _C)])
    plsc.subcore_barrier()

    base = (c * _NS + s) * _EPT

    pltpu.async_copy(col_hbm.at[pl.ds(base, _C)], idx_ca, sem_ica)
    pltpu.async_copy(row_hbm.at[pl.ds(base, _C)], idx_ra, sem_ira)
    pltpu.async_copy(eaw_hbm.at[pl.ds(base, _C)], eaw_v, sem_e)

    @pl.loop(0, _NBODY)
    def _body(t):
        offa = base + t * (2 * _C)
        offb = offa + _C
        offn = offb + _C
        pltpu.make_async_copy(col_hbm.at[pl.ds(offa, _C)], idx_ca,
                              sem_ica).wait()
        pltpu.make_async_copy(row_hbm.at[pl.ds(offa, _C)], idx_ra,
                              sem_ira).wait()

        @pl.when(t > 0)
        def _wsb():
            pltpu.make_async_copy(gath_b, acc_sh.at[idx_rb], sem_sb).wait()

        pltpu.async_copy(col_hbm.at[pl.ds(offb, _C)], idx_cb, sem_icb)
        pltpu.async_copy(row_hbm.at[pl.ds(offb, _C)], idx_rb, sem_irb)
        pltpu.async_copy(xw_hbm.at[idx_ca], gath_a, sem_ga)
        pltpu.make_async_copy(eaw_hbm.at[pl.ds(offa, _C)], eaw_v,
                              sem_e).wait()
        pltpu.make_async_copy(xw_hbm.at[idx_ca], gath_a, sem_ga).wait()
        _relu_add(gath_a, eaw_v, 0, _C)
        pltpu.async_copy(gath_a, acc_sh.at[idx_ra], sem_sa, add=True)
        pltpu.async_copy(eaw_hbm.at[pl.ds(offb, _C)], eaw_v, sem_e)
        pltpu.make_async_copy(col_hbm.at[pl.ds(offb, _C)], idx_cb,
                              sem_icb).wait()
        pltpu.make_async_copy(row_hbm.at[pl.ds(offb, _C)], idx_rb,
                              sem_irb).wait()
        pltpu.async_copy(xw_hbm.at[idx_cb], gath_b, sem_gb)
        pltpu.make_async_copy(eaw_hbm.at[pl.ds(offb, _C)], eaw_v,
                              sem_e).wait()
        pltpu.make_async_copy(xw_hbm.at[idx_cb], gath_b, sem_gb).wait()
        _relu_add(gath_b, eaw_v, 0, _C)
        pltpu.make_async_copy(gath_a, acc_sh.at[idx_ra], sem_sa).wait()

        @pl.when(t < _NBODY - 1)
        def _pref():
            pltpu.async_copy(col_hbm.at[pl.ds(offn, _C)], idx_ca, sem_ica)
            pltpu.async_copy(row_hbm.at[pl.ds(offn, _C)], idx_ra, sem_ira)
            pltpu.async_copy(eaw_hbm.at[pl.ds(offn, _C)], eaw_v, sem_e)

        pltpu.async_copy(gath_b, acc_sh.at[idx_rb], sem_sb, add=True)

    offt = base + 2 * _NBODY * _C
    pltpu.sync_copy(col_hbm.at[pl.ds(offt, _TAIL)], idx_tc)
    pltpu.sync_copy(row_hbm.at[pl.ds(offt, _TAIL)], idx_tr)
    pltpu.sync_copy(eaw_hbm.at[pl.ds(offt, _TAIL)],
                    eaw_v.at[pl.ds(0, _TAIL)])
    pltpu.async_copy(xw_hbm.at[idx_tc], gath_a.at[pl.ds(0, _TAIL)],
                     sem_ga).wait()
    _relu_add(gath_a, eaw_v, 0, _TAIL)
    pltpu.async_copy(gath_a.at[pl.ds(0, _TAIL)], acc_sh.at[idx_tr],
                     sem_sa, add=True)
    pltpu.make_async_copy(gath_b, acc_sh.at[idx_rb], sem_sb).wait()
    pltpu.make_async_copy(gath_a.at[pl.ds(0, _TAIL)], acc_sh.at[idx_tr],
                          sem_sa).wait()

    plsc.subcore_barrier()

    @pl.loop(s, _NWCHUNK, step=_NS)
    def _wout(t):
        off = t * _WR
        pltpu.sync_copy(acc_sh.at[pl.ds(off, _WR)],
                        gath_a.at[pl.ds(0, _WR)])
        pltpu.sync_copy(gath_a.at[pl.ds(0, _WR)],
                        sums_hbm.at[c, pl.ds(off, _WR)])


_sc_edge = functools.partial(
    pl.kernel,
    out_type=jax.ShapeDtypeStruct((_NC, N_NODES, HID), jnp.float32),
    mesh=plsc.VectorSubcoreMesh(core_axis_name="c", subcore_axis_name="s",
                                num_cores=_NC, num_subcores=_NS),
    scratch_types=[
        pltpu.VMEM((_C,), jnp.int32),
        pltpu.VMEM((_C,), jnp.int32),
        pltpu.VMEM((_C,), jnp.int32),
        pltpu.VMEM((_C,), jnp.int32),
        pltpu.VMEM((_TAIL,), jnp.int32),
        pltpu.VMEM((_TAIL,), jnp.int32),
        pltpu.VMEM((_C, HID), jnp.float32),
        pltpu.VMEM((_C, HID), jnp.float32),
        pltpu.VMEM((_C, HID), jnp.float32),
        pltpu.VMEM_SHARED((N_NODES, HID), jnp.float32),
        pltpu.SemaphoreType.DMA,
        pltpu.SemaphoreType.DMA,
        pltpu.SemaphoreType.DMA,
        pltpu.SemaphoreType.DMA,
        pltpu.SemaphoreType.DMA,
        pltpu.SemaphoreType.DMA,
        pltpu.SemaphoreType.DMA,
        pltpu.SemaphoreType.DMA,
        pltpu.SemaphoreType.DMA,
    ],
)(_sc_edge_body)


def _sc_cnt_body(row_hbm, cnts_hbm, idx_v, cnt_v):
    c = lax.axis_index("c")
    s = lax.axis_index("s")
    zero = jnp.zeros((16,), jnp.float32)
    one = jnp.ones((16,), jnp.float32)

    @pl.loop(0, N_NODES // 16)
    def _zcnt(i):
        cnt_v[pl.ds(i * 16, 16)] = zero

    base = (c * _NS + s) * _EPT

    @pl.loop(0, _EPT // _CC)
    def _chunk(g):
        pltpu.sync_copy(row_hbm.at[pl.ds(base + g * _CC, _CC)], idx_v)

        @pl.loop(0, _CC // 16)
        def _hist(j):
            iv = idx_v[pl.ds(j * 16, 16)]
            plsc.addupdate_scatter(cnt_v, [iv], one)

    pltpu.sync_copy(cnt_v, cnts_hbm.at[c * _NS + s])


_sc_cnt = functools.partial(
    pl.kernel,
    out_type=jax.ShapeDtypeStruct((_NC * _NS, N_NODES), jnp.float32),
    mesh=plsc.VectorSubcoreMesh(core_axis_name="c", subcore_axis_name="s",
                                num_cores=_NC, num_subcores=_NS),
    compiler_params=pltpu.CompilerParams(needs_layout_passes=False),
    scratch_types=[
        pltpu.VMEM((_CC,), jnp.int32),
        pltpu.VMEM((N_NODES,), jnp.float32),
    ],
)(_sc_cnt_body)


def _node_body(x_ref, s_ref, c_ref, w1b_ref, b1b_ref, w2a_ref, b2a_ref,
               w2b_ref, b2b_ref, o_ref):
    ssum = s_ref[0] + s_ref[1]
    cnt = jnp.sum(c_ref[...], axis=1, keepdims=True)
    m = ssum / jnp.maximum(cnt, 1.0)
    mean = jnp.dot(m, w1b_ref[...], preferred_element_type=jnp.float32)
    mean = jnp.where(cnt > 0.0, mean + b1b_ref[...], 0.0)
    h = jnp.dot(x_ref[...], w2a_ref[:D_FEAT, :],
                preferred_element_type=jnp.float32)
    h = h + jnp.dot(mean, w2a_ref[D_FEAT:, :],
                    preferred_element_type=jnp.float32)
    h = jnp.maximum(h + b2a_ref[...], 0.0)
    o_ref[...] = jnp.dot(h, w2b_ref[...],
                         preferred_element_type=jnp.float32) + b2b_ref[...]


def _node_call(x, sums, cnts, w1b, b1b, w2a, b2a, w2b, b2b):
    nb = _NB
    return pl.pallas_call(
        _node_body,
        grid=(N_NODES // nb,),
        in_specs=[
            pl.BlockSpec((nb, D_FEAT), lambda i: (i, 0)),
            pl.BlockSpec((_NC, nb, HID), lambda i: (0, i, 0)),
            pl.BlockSpec((nb, _NC * _NS), lambda i: (i, 0)),
            pl.BlockSpec((HID, HID), lambda i: (0, 0)),
            pl.BlockSpec((1, HID), lambda i: (0, 0)),
            pl.BlockSpec((HID + D_FEAT, HID), lambda i: (0, 0)),
            pl.BlockSpec((1, HID), lambda i: (0, 0)),
            pl.BlockSpec((HID, N_TGT), lambda i: (0, 0)),
            pl.BlockSpec((1, N_TGT), lambda i: (0, 0)),
        ],
        out_specs=pl.BlockSpec((nb, N_TGT), lambda i: (i, 0)),
        out_shape=jax.ShapeDtypeStruct((N_NODES, N_TGT), jnp.float32),
    )(x, sums, cnts, w1b, b1b.reshape(1, HID), w2a, b2a.reshape(1, HID),
      w2b, b2b.reshape(1, N_TGT))


def kernel(x, edge_index, edge_attr, u, batch,
           W1a, b1a, W1b, b1b, W2a, b2a, W2b, b2b):
    ei = edge_index.astype(jnp.int32)
    row = ei[0]
    col = ei[1]
    cnts = _sc_cnt(row)
    xw = _xw_call(x, W1a[:D_FEAT])
    eaw = _eaw_call(edge_attr, W1a[D_FEAT:], b1a)
    sums = _sc_edge(xw, eaw, col, row)
    cnts_t = jnp.transpose(cnts, (1, 0))
    return _node_call(x, sums, cnts_t, W1b, b1b, W2a, b2a, W2b, b2b)


# speedup vs baseline: 4.3670x; 1.1027x over previous
import functools

import jax
import jax.numpy as jnp
from jax import lax
from jax.experimental import pallas as pl
from jax.experimental.pallas import tpu as pltpu
from jax.experimental.pallas import tpu_sc as plsc

N_NODES = 10000
N_EDGES = 320000
D_FEAT = 128
D_EDGE = 16
HID = 128
N_TGT = 128

_NC, _NS = 2, 16
_C = 64
_EPT = N_EDGES // (_NC * _NS)
_NQ = 39
_TAIL = _EPT - 4 * _NQ * _C
_CC = 80
_WR = 80
_NWCHUNK = N_NODES // _WR
_EB = 2000
_NB = 1000


def _xw_body(x_ref, w_ref, o_ref):
    o_ref[...] = jnp.dot(x_ref[...], w_ref[...],
                         preferred_element_type=jnp.float32)


def _xw_call(x, w1a_x):
    return pl.pallas_call(
        _xw_body,
        out_shape=jax.ShapeDtypeStruct((N_NODES, HID), jnp.float32),
    )(x, w1a_x)


def _eaw_body(ea_ref, w_ref, b_ref, o_ref):
    o_ref[...] = jnp.dot(ea_ref[...], w_ref[...],
                         preferred_element_type=jnp.float32) + b_ref[...]


def _eaw_call(ea, w1a_e, b1a):
    return pl.pallas_call(
        _eaw_body,
        grid=(N_EDGES // _EB,),
        in_specs=[
            pl.BlockSpec((_EB, D_EDGE), lambda i: (i, 0)),
            pl.BlockSpec((D_EDGE, HID), lambda i: (0, 0)),
            pl.BlockSpec((1, HID), lambda i: (0, 0)),
        ],
        out_specs=pl.BlockSpec((_EB, HID), lambda i: (i, 0)),
        out_shape=jax.ShapeDtypeStruct((N_EDGES, HID), jnp.float32),
    )(ea, w1a_e, b1a.reshape(1, HID))


def _relu_add(gath, eaw, lo, n):
    @pl.loop(lo, lo + n)
    def _row(r):
        for k in range(HID // 16):
            sl = pl.ds(k * 16, 16)
            gath[r, sl] = jnp.maximum(gath[r, sl] + eaw[r, sl], 0.0)


def _sc_edge_body(xw_hbm, eaw_hbm, col_hbm, row_hbm, sums_hbm,
                  idx_c0, idx_r0, idx_c1, idx_r1,
                  idx_c2, idx_r2, idx_c3, idx_r3, idx_tc, idx_tr,
                  eaw_v, gath_a, gath_b, acc_sh,
                  sem_ic0, sem_ir0, sem_ic1, sem_ir1,
                  sem_ic2, sem_ir2, sem_ic3, sem_ir3,
                  sem_ga, sem_gb, sem_e, sem_sa, sem_sb):
    c = lax.axis_index("c")
    s = lax.axis_index("s")
    zero = jnp.zeros((16,), jnp.float32)
    idx_c = [idx_c0, idx_c1, idx_c2, idx_c3]
    idx_r = [idx_r0, idx_r1, idx_r2, idx_r3]
    sem_ic = [sem_ic0, sem_ic1, sem_ic2, sem_ic3]
    sem_ir = [sem_ir0, sem_ir1, sem_ir2, sem_ir3]
    gath = [gath_a, gath_b]
    sem_g = [sem_ga, sem_gb]
    sem_s = [sem_sa, sem_sb]

    @pl.loop(0, _C)
    def _zrow(i):
        for k in range(HID // 16):
            gath_a[i, pl.ds(k * 16, 16)] = zero

    @pl.loop(s, N_NODES // _C, step=_NS)
    def _zinit(t):
        pltpu.sync_copy(gath_a, acc_sh.at[pl.ds(t * _C, _C)])
    plsc.subcore_barrier()

    base = (c * _NS + s) * _EPT

    def idx_issue(off, k):
        pltpu.async_copy(col_hbm.at[pl.ds(off, _C)], idx_c[k], sem_ic[k])
        pltpu.async_copy(row_hbm.at[pl.ds(off, _C)], idx_r[k], sem_ir[k])

    def idx_wait(off, k):
        pltpu.make_async_copy(col_hbm.at[pl.ds(off, _C)], idx_c[k],
                              sem_ic[k]).wait()
        pltpu.make_async_copy(row_hbm.at[pl.ds(off, _C)], idx_r[k],
                              sem_ir[k]).wait()

    def eaw_issue(off):
        pltpu.async_copy(eaw_hbm.at[pl.ds(off, _C)], eaw_v, sem_e)

    def eaw_wait(off):
        pltpu.make_async_copy(eaw_hbm.at[pl.ds(off, _C)], eaw_v,
                              sem_e).wait()

    def g_issue(k, m):
        pltpu.async_copy(xw_hbm.at[idx_c[k]], gath[m], sem_g[m])

    def g_wait(k, m):
        pltpu.make_async_copy(xw_hbm.at[idx_c[k]], gath[m], sem_g[m]).wait()

    def s_issue(k, m):
        pltpu.async_copy(gath[m], acc_sh.at[idx_r[k]], sem_s[m], add=True)

    def s_wait(k, m):
        pltpu.make_async_copy(gath[m], acc_sh.at[idx_r[k]], sem_s[m]).wait()

    for k in range(4):
        idx_issue(base + k * _C, k)
    eaw_issue(base)
    idx_wait(base, 0)
    g_issue(0, 0)

    @pl.loop(0, _NQ)
    def _body(q):
        g0 = base + q * (4 * _C)
        g1, g2, g3 = g0 + _C, g0 + 2 * _C, g0 + 3 * _C

        @pl.when(q > 0)
        def _w0():
            s_wait(3, 1)
            idx_issue(g3, 3)
        idx_wait(g1, 1)
        g_issue(1, 1)
        eaw_wait(g0)
        g_wait(0, 0)
        _relu_add(gath_a, eaw_v, 0, _C)
        s_issue(0, 0)
        eaw_issue(g1)

        s_wait(0, 0)

        @pl.when(q < _NQ - 1)
        def _p0():
            idx_issue(g0 + 4 * _C, 0)
        idx_wait(g2, 2)
        g_issue(2, 0)
        eaw_wait(g1)
        g_wait(1, 1)
        _relu_add(gath_b, eaw_v, 0, _C)
        s_issue(1, 1)
        eaw_issue(g2)

        s_wait(1, 1)

        @pl.when(q < _NQ - 1)
        def _p1():
            idx_issue(g1 + 4 * _C, 1)
        idx_wait(g3, 3)
        g_issue(3, 1)
        eaw_wait(g2)
        g_wait(2, 0)
        _relu_add(gath_a, eaw_v, 0, _C)
        s_issue(2, 0)
        eaw_issue(g3)

        s_wait(2, 0)

        @pl.when(q < _NQ - 1)
        def _p2():
            idx_issue(g2 + 4 * _C, 2)
            idx_wait(g0 + 4 * _C, 0)
            g_issue(0, 0)
        eaw_wait(g3)
        g_wait(3, 1)
        _relu_add(gath_b, eaw_v, 0, _C)
        s_issue(3, 1)

        @pl.when(q < _NQ - 1)
        def _p3():
            eaw_issue(g0 + 4 * _C)

    offt = base + 4 * _NQ * _C
    pltpu.sync_copy(col_hbm.at[pl.ds(offt, _TAIL)], idx_tc)
    pltpu.sync_copy(row_hbm.at[pl.ds(offt, _TAIL)], idx_tr)
    pltpu.sync_copy(eaw_hbm.at[pl.ds(offt, _TAIL)],
                    eaw_v.at[pl.ds(0, _TAIL)])
    pltpu.async_copy(xw_hbm.at[idx_tc], gath_a.at[pl.ds(0, _TAIL)],
                     sem_ga).wait()
    _relu_add(gath_a, eaw_v, 0, _TAIL)
    pltpu.async_copy(gath_a.at[pl.ds(0, _TAIL)], acc_sh.at[idx_tr],
                     sem_sa, add=True)
    s_wait(3, 1)
    pltpu.make_async_copy(gath_a.at[pl.ds(0, _TAIL)], acc_sh.at[idx_tr],
                          sem_sa).wait()

    plsc.subcore_barrier()

    @pl.loop(s, _NWCHUNK, step=_NS)
    def _wout(t):
        off = t * _WR
        pltpu.sync_copy(acc_sh.at[pl.ds(off, _WR)],
                        gath_a.at[pl.ds(0, _WR)])
        pltpu.sync_copy(gath_a.at[pl.ds(0, _WR)],
                        sums_hbm.at[c, pl.ds(off, _WR)])


_sc_edge = functools.partial(
    pl.kernel,
    out_type=jax.ShapeDtypeStruct((_NC, N_NODES, HID), jnp.float32),
    mesh=plsc.VectorSubcoreMesh(core_axis_name="c", subcore_axis_name="s",
                                num_cores=_NC, num_subcores=_NS),
    scratch_types=(
        [pltpu.VMEM((_C,), jnp.int32) for _ in range(8)]
        + [pltpu.VMEM((_TAIL,), jnp.int32) for _ in range(2)]
        + [
            pltpu.VMEM((_C, HID), jnp.float32),
            pltpu.VMEM((_C, HID), jnp.float32),
            pltpu.VMEM((_C, HID), jnp.float32),
            pltpu.VMEM_SHARED((N_NODES, HID), jnp.float32),
        ]
        + [pltpu.SemaphoreType.DMA for _ in range(13)]
    ),
)(_sc_edge_body)


def _sc_cnt_body(row_hbm, cnts_hbm, idx_v, cnt_v):
    c = lax.axis_index("c")
    s = lax.axis_index("s")
    zero = jnp.zeros((16,), jnp.float32)
    one = jnp.ones((16,), jnp.float32)

    @pl.loop(0, N_NODES // 16)
    def _zcnt(i):
        cnt_v[pl.ds(i * 16, 16)] = zero

    base = (c * _NS + s) * _EPT

    @pl.loop(0, _EPT // _CC)
    def _chunk(g):
        pltpu.sync_copy(row_hbm.at[pl.ds(base + g * _CC, _CC)], idx_v)

        @pl.loop(0, _CC // 16)
        def _hist(j):
            iv = idx_v[pl.ds(j * 16, 16)]
            plsc.addupdate_scatter(cnt_v, [iv], one)

    pltpu.sync_copy(cnt_v, cnts_hbm.at[c * _NS + s])


_sc_cnt = functools.partial(
    pl.kernel,
    out_type=jax.ShapeDtypeStruct((_NC * _NS, N_NODES), jnp.float32),
    mesh=plsc.VectorSubcoreMesh(core_axis_name="c", subcore_axis_name="s",
                                num_cores=_NC, num_subcores=_NS),
    compiler_params=pltpu.CompilerParams(needs_layout_passes=False),
    scratch_types=[
        pltpu.VMEM((_CC,), jnp.int32),
        pltpu.VMEM((N_NODES,), jnp.float32),
    ],
)(_sc_cnt_body)


def _node_body(x_ref, s_ref, c_ref, w1b_ref, b1b_ref, w2a_ref, b2a_ref,
               w2b_ref, b2b_ref, o_ref):
    ssum = s_ref[0] + s_ref[1]
    cnt = jnp.sum(c_ref[...], axis=1, keepdims=True)
    m = ssum / jnp.maximum(cnt, 1.0)
    mean = jnp.dot(m, w1b_ref[...], preferred_element_type=jnp.float32)
    mean = jnp.where(cnt > 0.0, mean + b1b_ref[...], 0.0)
    h = jnp.dot(x_ref[...], w2a_ref[:D_FEAT, :],
                preferred_element_type=jnp.float32)
    h = h + jnp.dot(mean, w2a_ref[D_FEAT:, :],
                    preferred_element_type=jnp.float32)
    h = jnp.maximum(h + b2a_ref[...], 0.0)
    o_ref[...] = jnp.dot(h, w2b_ref[...],
                         preferred_element_type=jnp.float32) + b2b_ref[...]


def _node_call(x, sums, cnts, w1b, b1b, w2a, b2a, w2b, b2b):
    nb = _NB
    return pl.pallas_call(
        _node_body,
        grid=(N_NODES // nb,),
        in_specs=[
            pl.BlockSpec((nb, D_FEAT), lambda i: (i, 0)),
            pl.BlockSpec((_NC, nb, HID), lambda i: (0, i, 0)),
            pl.BlockSpec((nb, _NC * _NS), lambda i: (i, 0)),
            pl.BlockSpec((HID, HID), lambda i: (0, 0)),
            pl.BlockSpec((1, HID), lambda i: (0, 0)),
            pl.BlockSpec((HID + D_FEAT, HID), lambda i: (0, 0)),
            pl.BlockSpec((1, HID), lambda i: (0, 0)),
            pl.BlockSpec((HID, N_TGT), lambda i: (0, 0)),
            pl.BlockSpec((1, N_TGT), lambda i: (0, 0)),
        ],
        out_specs=pl.BlockSpec((nb, N_TGT), lambda i: (i, 0)),
        out_shape=jax.ShapeDtypeStruct((N_NODES, N_TGT), jnp.float32),
    )(x, sums, cnts, w1b, b1b.reshape(1, HID), w2a, b2a.reshape(1, HID),
      w2b, b2b.reshape(1, N_TGT))


def kernel(x, edge_index, edge_attr, u, batch,
           W1a, b1a, W1b, b1b, W2a, b2a, W2b, b2b):
    ei = edge_index.astype(jnp.int32)
    row = ei[0]
    col = ei[1]
    cnts = _sc_cnt(row)
    xw = _xw_call(x, W1a[:D_FEAT])
    eaw = _eaw_call(edge_attr, W1a[D_FEAT:], b1a)
    sums = _sc_edge(xw, eaw, col, row)
    cnts_t = jnp.transpose(cnts, (1, 0))
    return _node_call(x, sums, cnts_t, W1b, b1b, W2a, b2a, W2b, b2b)
